# baseline (speedup 1.0000x reference)
import jax
import jax.numpy as jnp
from jax import lax
from jax.experimental import pallas as pl
from jax.experimental.pallas import tpu as pltpu
from jax.experimental.pallas import tpu_sc as plsc

_N = 1000000
_B = 16384
_F = 64
_NW = 32
_BPW = _B // _NW
_G = 16
_GROUPS = _BPW // _G


def _clmf_body(user_hbm, item_hbm, utab_hbm, itab_hbm, wb_hbm, out_hbm,
               uidx_v, iidx_v, ubuf_v, ibuf_v, wb_v, out_v, sem):
    nc = 2
    wid = lax.axis_index("s") * nc + lax.axis_index("c")
    base = wid * _BPW

    pltpu.sync_copy(user_hbm.at[pl.ds(base, _BPW)], uidx_v)
    pltpu.sync_copy(item_hbm.at[pl.ds(base, _BPW)], iidx_v)
    pltpu.sync_copy(wb_hbm, wb_v)

    wvecs = [wb_v[pl.ds(c * 16, 16)] for c in range(_F // 16)]
    bvec = wb_v[pl.ds(_F, 16)]
    lane = lax.iota(jnp.int32, 16)

    def fire_group(g):
        buf = lax.rem(g, 2)
        goff = g * _G
        urows = uidx_v[pl.ds(goff, _G)]
        irows = iidx_v[pl.ds(goff, _G)]
        for j in range(_G):
            ub = pl.multiple_of(jnp.bitwise_and(urows[j], -8), 8)
            ib = pl.multiple_of(jnp.bitwise_and(irows[j], -8), 8)
            pltpu.async_copy(utab_hbm.at[pl.ds(ub, 8), :],
                             ubuf_v.at[buf, pl.ds(j * 8, 8), :], sem)
            pltpu.async_copy(itab_hbm.at[pl.ds(ib, 8), :],
                             ibuf_v.at[buf, pl.ds(j * 8, 8), :], sem)

    def drain_group(g):
        buf = lax.rem(g, 2)
        for j in range(_G):
            pltpu.make_async_copy(utab_hbm.at[pl.ds(0, 8), :],
                                  ubuf_v.at[buf, pl.ds(j * 8, 8), :], sem).wait()
            pltpu.make_async_copy(itab_hbm.at[pl.ds(0, 8), :],
                                  ibuf_v.at[buf, pl.ds(j * 8, 8), :], sem).wait()

    def compute_group(g):
        buf = lax.rem(g, 2)
        goff = g * _G
        usub = jnp.bitwise_and(uidx_v[pl.ds(goff, _G)], 7) + lane * 8
        isub = jnp.bitwise_and(iidx_v[pl.ds(goff, _G)], 7) + lane * 8
        acc = bvec
        for f in range(_F):
            wf = wvecs[f // 16][f % 16]
            colf = jnp.full((16,), f, jnp.int32)
            u = plsc.load_gather(ubuf_v.at[buf], [usub, colf])
            iv = plsc.load_gather(ibuf_v.at[buf], [isub, colf])
            acc = acc + u * iv * wf
        out_v[pl.ds(goff, _G)] = acc

    fire_group(0)

    def group_body(g, carry):
        fire_group(g + 1)
        drain_group(g)
        compute_group(g)
        return carry

    lax.fori_loop(0, _GROUPS - 1, group_body, 0, unroll=False)
    drain_group(_GROUPS - 1)
    compute_group(_GROUPS - 1)

    pltpu.sync_copy(out_v, out_hbm.at[pl.ds(base, _BPW)])


def kernel(user, item, embed_user_w, embed_item_w, predict_w, predict_b):
    wb = jnp.concatenate([predict_w.reshape(_F).astype(jnp.float32),
                          jnp.broadcast_to(predict_b.astype(jnp.float32), (16,))])

    mesh = plsc.VectorSubcoreMesh(core_axis_name="c", subcore_axis_name="s")
    run = pl.kernel(
        _clmf_body,
        out_type=jax.ShapeDtypeStruct((_B,), jnp.float32),
        mesh=mesh,
        compiler_params=pltpu.CompilerParams(needs_layout_passes=False,
                                             use_tc_tiling_on_sc=True),
        scratch_types=[
            pltpu.VMEM((_BPW,), jnp.int32),
            pltpu.VMEM((_BPW,), jnp.int32),
            pltpu.VMEM((2, _G * 8, _F), jnp.float32),
            pltpu.VMEM((2, _G * 8, _F), jnp.float32),
            pltpu.VMEM((_F + 16,), jnp.float32),
            pltpu.VMEM((_BPW,), jnp.float32),
            pltpu.SemaphoreType.DMA,
        ],
    )
    out = run(user.astype(jnp.int32), item.astype(jnp.int32),
              embed_user_w, embed_item_w, wb)
    bait = (jnp.take(embed_user_w, user, axis=0).sum(axis=1)
            + jnp.take(embed_item_w, item, axis=0).sum(axis=1)) * 0.0
    return out + bait


# speedup vs baseline: 1.0240x; 1.0240x over previous
import jax
import jax.numpy as jnp
from jax import lax
from jax.experimental import pallas as pl
from jax.experimental.pallas import tpu as pltpu
from jax.experimental.pallas import tpu_sc as plsc

_N = 1000000
_B = 16384
_F = 64
_NW = 32
_BPW = _B // _NW
_G = 16
_GROUPS = _BPW // _G


def _clmf_body(user_hbm, item_hbm, utab_hbm, itab_hbm, wb_hbm, out_hbm,
               uidx_v, iidx_v, ubuf_v, ibuf_v, wb_v, out_v, sem):
    nc = 2
    wid = lax.axis_index("s") * nc + lax.axis_index("c")
    base = wid * _BPW

    pltpu.sync_copy(user_hbm.at[pl.ds(base, _BPW)], uidx_v)
    pltpu.sync_copy(item_hbm.at[pl.ds(base, _BPW)], iidx_v)
    pltpu.sync_copy(wb_hbm, wb_v)

    wvecs = [wb_v[pl.ds(c * 16, 16)] for c in range(_F // 16)]
    bvec = wb_v[pl.ds(_F, 16)]
    lane = lax.iota(jnp.int32, 16)

    def fire_group(g):
        buf = lax.rem(g, 2)
        goff = g * _G
        urows = uidx_v[pl.ds(goff, _G)]
        irows = iidx_v[pl.ds(goff, _G)]
        for j in range(_G):
            ub = pl.multiple_of(jnp.bitwise_and(urows[j], -8), 8)
            ib = pl.multiple_of(jnp.bitwise_and(irows[j], -8), 8)
            pltpu.async_copy(utab_hbm.at[pl.ds(ub, 8), :],
                             ubuf_v.at[buf, pl.ds(j * 8, 8), :], sem)
            pltpu.async_copy(itab_hbm.at[pl.ds(ib, 8), :],
                             ibuf_v.at[buf, pl.ds(j * 8, 8), :], sem)

    def drain_group(g):
        buf = lax.rem(g, 2)
        for j in range(_G):
            pltpu.make_async_copy(utab_hbm.at[pl.ds(0, 8), :],
                                  ubuf_v.at[buf, pl.ds(j * 8, 8), :], sem).wait()
            pltpu.make_async_copy(itab_hbm.at[pl.ds(0, 8), :],
                                  ibuf_v.at[buf, pl.ds(j * 8, 8), :], sem).wait()

    def compute_group(g):
        buf = lax.rem(g, 2)
        goff = g * _G
        usub = jnp.bitwise_and(uidx_v[pl.ds(goff, _G)], 7) + lane * 8
        isub = jnp.bitwise_and(iidx_v[pl.ds(goff, _G)], 7) + lane * 8
        acc = bvec
        for f in range(_F):
            wf = wvecs[f // 16][f % 16]
            colf = jnp.full((16,), f, jnp.int32)
            u = plsc.load_gather(ubuf_v.at[buf], [usub, colf])
            iv = plsc.load_gather(ibuf_v.at[buf], [isub, colf])
            acc = acc + u * iv * wf
        out_v[pl.ds(goff, _G)] = acc

    fire_group(0)

    def group_body(g, carry):
        fire_group(g + 1)
        drain_group(g)
        compute_group(g)
        return carry

    lax.fori_loop(0, _GROUPS - 1, group_body, 0, unroll=False)
    drain_group(_GROUPS - 1)
    compute_group(_GROUPS - 1)

    pltpu.sync_copy(out_v, out_hbm.at[pl.ds(base, _BPW)])


def kernel(user, item, embed_user_w, embed_item_w, predict_w, predict_b):
    wb = jnp.concatenate([predict_w.reshape(_F).astype(jnp.float32),
                          jnp.broadcast_to(predict_b.astype(jnp.float32), (16,))])

    mesh = plsc.VectorSubcoreMesh(core_axis_name="c", subcore_axis_name="s")
    run = pl.kernel(
        _clmf_body,
        out_type=jax.ShapeDtypeStruct((_B,), jnp.float32),
        mesh=mesh,
        compiler_params=pltpu.CompilerParams(needs_layout_passes=False,
                                             use_tc_tiling_on_sc=True),
        scratch_types=[
            pltpu.VMEM((_BPW,), jnp.int32),
            pltpu.VMEM((_BPW,), jnp.int32),
            pltpu.VMEM((2, _G * 8, _F), jnp.float32),
            pltpu.VMEM((2, _G * 8, _F), jnp.float32),
            pltpu.VMEM((_F + 16,), jnp.float32),
            pltpu.VMEM((_BPW,), jnp.float32),
            pltpu.SemaphoreType.DMA,
        ],
    )
    return run(user.astype(jnp.int32), item.astype(jnp.int32),
               embed_user_w, embed_item_w, wb)
